# baseline (speedup 1.0000x reference)
import jax
import jax.numpy as jnp
import numpy as np
from jax import lax
from jax.experimental import pallas as pl
from jax.experimental.pallas import tpu as pltpu
from jax.experimental.pallas import tpu_sc as plsc

NC = 2
NS = 16
LANES = 16

N = 10000
D = 128
E = 320000

NW = NC * NS
EW = E // NW
C = 40
NCH = EW // C

EW2 = E // NS
NCH2 = EW2 // C

RB = 1000
NRW = N // RB
DBLK = 2000

_f32 = jnp.float32


def _deg_body(dst2_hbm, ew2_hbm, degp_hbm, deg_hbm,
              didx_all, wv_all, degv, dbuf, tmp2, sem):
    c = lax.axis_index("c")
    s = lax.axis_index("s")

    @pl.when(c == 0)
    def _():
        def zero16(i, _):
            degv[pl.ds(i * LANES, LANES)] = jnp.zeros((LANES,), _f32)
            return 0
        lax.fori_loop(0, N // LANES, zero16, 0)

        pltpu.sync_copy(dst2_hbm.at[s], didx_all)
        pltpu.sync_copy(ew2_hbm.at[s], wv_all)

        def grp(i, _):
            sl = pl.ds(i * LANES, LANES)
            plsc.addupdate_scatter(degv, [didx_all[sl]], wv_all[sl])
            return 0
        lax.fori_loop(0, EW2 // LANES, grp, 0)
        pltpu.sync_copy(degv, degp_hbm.at[pl.ds(s * N, N)])

    plsc.subcore_barrier()

    @pl.when((c == 0) & (s < N // DBLK))
    def _():
        copies = [
            pltpu.async_copy(degp_hbm.at[pl.ds(p * N + s * DBLK, DBLK)],
                             tmp2.at[p], sem)
            for p in range(NS)
        ]
        for cp in copies:
            cp.wait()

        def addv(i, _):
            sl = pl.ds(i * LANES, LANES)
            acc = tmp2[0, sl]
            for p in range(1, NS):
                acc = acc + tmp2[p, sl]
            dbuf[sl] = acc
            return 0
        lax.fori_loop(0, DBLK // LANES, addv, 0)
        pltpu.sync_copy(dbuf, deg_hbm.at[pl.ds(s * DBLK, DBLK)])


_SC_PARAMS = pltpu.CompilerParams(
    use_tc_tiling_on_sc=False, needs_layout_passes=False)

_sc_deg = pl.kernel(
    _deg_body,
    out_type=(jax.ShapeDtypeStruct((NS * N,), _f32),
              jax.ShapeDtypeStruct((N,), _f32)),
    compiler_params=_SC_PARAMS,
    mesh=plsc.VectorSubcoreMesh(core_axis_name="c", subcore_axis_name="s"),
    scratch_types=[
        pltpu.VMEM((EW2,), jnp.int32),
        pltpu.VMEM((EW2,), _f32),
        pltpu.VMEM((N,), _f32),
        pltpu.VMEM((DBLK,), _f32),
        pltpu.VMEM((NS, DBLK), _f32),
        pltpu.SemaphoreType.DMA,
    ],
)


def _spmm_body(zb_hbm, src3_hbm, dst3_hbm, ew3_hbm, zeros_hbm, out_hbm,
               sidx_all, didx_all, wv_all,
               rows0, rows1, rows2, rows3, msg0, msg1,
               acc_sh,
               gsem0, gsem1, gsem2, gsem3,
               ssem0, ssem1):
    c = lax.axis_index("c")
    s = lax.axis_index("s")
    wid = c * NS + s
    rows = (rows0, rows1, rows2, rows3)
    msg = (msg0, msg1)
    gsem = (gsem0, gsem1, gsem2, gsem3)
    ssem = (ssem0, ssem1)

    @pl.when(s < NRW)
    def _():
        pltpu.sync_copy(zeros_hbm.at[pl.ds(s * RB, RB)],
                        acc_sh.at[pl.ds(s * RB, RB)])
    pltpu.sync_copy(src3_hbm.at[wid], sidx_all)
    pltpu.sync_copy(dst3_hbm.at[wid], didx_all)
    pltpu.sync_copy(ew3_hbm.at[wid], wv_all)
    plsc.subcore_barrier()

    def gather(ci, sl):
        pltpu.async_copy(zb_hbm.at[sidx_all.at[ci]], rows[sl], gsem[sl])

    def gwait(sl):
        pltpu.make_async_copy(zb_hbm.at[sidx_all.at[0]], rows[sl],
                              gsem[sl]).wait()

    def scat(ci, mi):
        pltpu.async_copy(msg[mi], acc_sh.at[didx_all.at[ci]], ssem[mi],
                         add=True)

    def swait(mi):
        pltpu.make_async_copy(msg[mi], acc_sh.at[didx_all.at[0]],
                              ssem[mi]).wait()

    def scale(ci, sl, mi):
        buf = rows[sl]
        m = msg[mi]
        cvec = jnp.full((LANES,), ci, jnp.int32)

        @plsc.parallel_loop(0, C, unroll=8)
        def _(r):
            wb = plsc.load_gather(
                wv_all, [cvec, jnp.full((LANES,), r, jnp.int32)])
            for g in range(D // 32):
                x32 = buf[r, pl.ds(32 * g, 32)]
                a, b = plsc.unpack(x32, format=plsc.PackFormat.INTERLEAVED)
                m[r, pl.ds(32 * g, LANES)] = a * wb
                m[r, pl.ds(32 * g + LANES, LANES)] = b * wb

    gather(0, 0)
    gather(1, 1)
    gather(2, 2)

    def step(k, _):
        for off in range(4):
            ci = 4 * k + off
            mi = off % 2
            gwait(off)
            if off == 3:
                @pl.when(ci + 3 < NCH)
                def _(ci=ci, off=off):
                    gather(ci + 3, (off + 3) % 4)
            else:
                gather(ci + 3, (off + 3) % 4)
            if off >= 2:
                swait(mi)
            else:
                @pl.when(k > 0)
                def _(mi=mi):
                    swait(mi)
            scale(ci, off, mi)
            scat(ci, mi)
        return 0
    lax.fori_loop(0, NCH // 4, step, 0)

    for ci, off in ((NCH - 2, 0), (NCH - 1, 1)):
        gwait(off)
        swait(off % 2)
        scale(ci, off, off % 2)
        scat(ci, off % 2)
    swait(0)
    swait(1)

    plsc.subcore_barrier()

    @pl.when(s < NRW)
    def _():
        pltpu.sync_copy(acc_sh.at[pl.ds(s * RB, RB)],
                        out_hbm.at[pl.ds(c * N + s * RB, RB)])


assert NCH % 4 == 2

_sc_spmm = pl.kernel(
    _spmm_body,
    out_type=jax.ShapeDtypeStruct((NC * N, D), _f32),
    compiler_params=_SC_PARAMS,
    mesh=plsc.VectorSubcoreMesh(core_axis_name="c", subcore_axis_name="s"),
    scratch_types=[
        pltpu.VMEM((NCH, C), jnp.int32),
        pltpu.VMEM((NCH, C), jnp.int32),
        pltpu.VMEM((NCH, C), _f32),
        pltpu.VMEM((C, D), jnp.bfloat16),
        pltpu.VMEM((C, D), jnp.bfloat16),
        pltpu.VMEM((C, D), jnp.bfloat16),
        pltpu.VMEM((C, D), jnp.bfloat16),
        pltpu.VMEM((C, D), _f32),
        pltpu.VMEM((C, D), _f32),
        pltpu.VMEM_SHARED((N, D), _f32),
        pltpu.SemaphoreType.DMA,
        pltpu.SemaphoreType.DMA,
        pltpu.SemaphoreType.DMA,
        pltpu.SemaphoreType.DMA,
        pltpu.SemaphoreType.DMA,
        pltpu.SemaphoreType.DMA,
    ],
)


BR = 10000
G = N // BR


def _tc1_body(deg_ref, x_ref, W_ref, Wp_ref, Wr_ref, br_ref,
              z_ref, zb_ref, id_ref):
    dis = lax.rsqrt(deg_ref[...] + 1.0)
    xb = x_ref[...]
    z_ref[...] = jnp.dot(xb, W_ref[...], preferred_element_type=_f32) * dis
    zb_ref[...] = (jnp.dot(xb, Wp_ref[...], preferred_element_type=_f32)
                   * dis).astype(jnp.bfloat16)
    id_ref[...] = jnp.dot(xb, Wr_ref[...], preferred_element_type=_f32) + br_ref[...]


def _ln(h, g, be):
    mu = jnp.mean(h, axis=-1, keepdims=True)
    var = jnp.mean((h - mu) ** 2, axis=-1, keepdims=True)
    return (h - mu) * lax.rsqrt(var + 1e-5) * g + be


def _tc_post_body(acc_ref, z_ref, id_ref, deg_ref, b_ref, g_ref, be_ref,
                  Wn_ref, Wpn_ref, Wrn_ref, brn_ref,
                  zn_ref, zbn_ref, idn_ref):
    dis = lax.rsqrt(deg_ref[...] + 1.0)
    out = (acc_ref[0] + acc_ref[1] + z_ref[...]) * dis + b_ref[...] + id_ref[...]
    h = _ln(out, g_ref[...], be_ref[...])
    h = h * jax.nn.sigmoid(h)
    zn_ref[...] = jnp.dot(h, Wn_ref[...], preferred_element_type=_f32) * dis
    zbn_ref[...] = (jnp.dot(h, Wpn_ref[...], preferred_element_type=_f32)
                    * dis).astype(jnp.bfloat16)
    idn_ref[...] = jnp.dot(h, Wrn_ref[...], preferred_element_type=_f32) + brn_ref[...]


def _tc_final_body(acc_ref, z_ref, id_ref, deg_ref, b_ref, g_ref, be_ref,
                   res_ref):
    i = pl.program_id(0)
    dis = lax.rsqrt(deg_ref[...] + 1.0)
    out = (acc_ref[0] + acc_ref[1] + z_ref[...]) * dis + b_ref[...] + id_ref[...]
    h = _ln(out, g_ref[...], be_ref[...])
    part = jnp.sum(h, axis=0, keepdims=True) * (1.0 / N)

    @pl.when(i == 0)
    def _():
        res_ref[...] = jnp.zeros_like(res_ref)
    res_ref[...] += part


def _row_spec(last):
    return pl.BlockSpec((BR, last), lambda i: (i, 0))


def _full_spec(shape):
    nd = len(shape)
    return pl.BlockSpec(shape, lambda i: (0,) * nd)


def _tc1(deg, x, W, Wp, Wr, br):
    return pl.pallas_call(
        _tc1_body,
        grid=(G,),
        in_specs=[_row_spec(1), _row_spec(D), _full_spec((D, D)),
                  _full_spec((D, D)), _full_spec((D, D)), _full_spec((1, D))],
        out_specs=[_row_spec(D), _row_spec(D), _row_spec(D)],
        out_shape=[jax.ShapeDtypeStruct((N, D), _f32),
                   jax.ShapeDtypeStruct((N, D), jnp.bfloat16),
                   jax.ShapeDtypeStruct((N, D), _f32)],
    )(deg, x, W, Wp, Wr, br)


def _tc_post(acc, z, idn, deg, b, g, be, Wn, Wpn, Wrn, brn):
    return pl.pallas_call(
        _tc_post_body,
        grid=(G,),
        in_specs=[pl.BlockSpec((NC, BR, D), lambda i: (0, i, 0)),
                  _row_spec(D), _row_spec(D), _row_spec(1),
                  _full_spec((1, D)), _full_spec((1, D)), _full_spec((1, D)),
                  _full_spec((D, D)), _full_spec((D, D)), _full_spec((D, D)),
                  _full_spec((1, D))],
        out_specs=[_row_spec(D), _row_spec(D), _row_spec(D)],
        out_shape=[jax.ShapeDtypeStruct((N, D), _f32),
                   jax.ShapeDtypeStruct((N, D), jnp.bfloat16),
                   jax.ShapeDtypeStruct((N, D), _f32)],
    )(acc, z, idn, deg, b, g, be, Wn, Wpn, Wrn, brn)


def _tc_final(acc, z, idn, deg, b, g, be):
    return pl.pallas_call(
        _tc_final_body,
        grid=(G,),
        in_specs=[pl.BlockSpec((NC, BR, D), lambda i: (0, i, 0)),
                  _row_spec(D), _row_spec(D), _row_spec(1),
                  _full_spec((1, D)), _full_spec((1, D)), _full_spec((1, D))],
        out_specs=pl.BlockSpec((1, D), lambda i: (0, 0)),
        out_shape=jax.ShapeDtypeStruct((1, D), _f32),
    )(acc, z, idn, deg, b, g, be)


def kernel(x, edge_index, edge_weight,
           W1, b1, Wr1, br1, g1, be1,
           W2, b2, Wr2, br2, g2, be2,
           W3, b3, Wr3, br3, g3, be3):
    src = edge_index[0].astype(jnp.int32)
    dst = edge_index[1].astype(jnp.int32)
    ew = edge_weight
    zeros = jnp.zeros((N, D), _f32)

    src3 = src.reshape(NW, NCH, C)
    dst3 = dst.reshape(NW, NCH, C)
    ew3 = ew.reshape(NW, NCH, C)
    dst2 = dst.reshape(NS, EW2)
    ew2 = ew.reshape(NS, EW2)

    _, deg_raw = _sc_deg(dst2, ew2)
    deg = deg_raw.reshape(N, 1)

    def r1(v):
        return v.reshape(1, D)

    def spmm(zb):
        return _sc_spmm(zb, src3, dst3, ew3, zeros).reshape(NC, N, D)

    perm = np.empty((D,), np.int32)
    for g in range(D // 32):
        for i in range(16):
            perm[32 * g + 2 * i] = 32 * g + i
            perm[32 * g + 2 * i + 1] = 32 * g + 16 + i
    W1p, W2p, W3p = W1[:, perm], W2[:, perm], W3[:, perm]

    z, zb, idn = _tc1(deg, x, W1, W1p, Wr1, r1(br1))
    acc = spmm(zb)
    z, zb, idn = _tc_post(acc, z, idn, deg, r1(b1), r1(g1), r1(be1),
                          W2, W2p, Wr2, r1(br2))
    acc = spmm(zb)
    z, zb, idn = _tc_post(acc, z, idn, deg, r1(b2), r1(g2), r1(be2),
                          W3, W3p, Wr3, r1(br3))
    acc = spmm(zb)
    return _tc_final(acc, z, idn, deg, r1(b3), r1(g3), r1(be3))


# speedup vs baseline: 1.0220x; 1.0220x over previous
import jax
import jax.numpy as jnp
import numpy as np
from jax import lax
from jax.experimental import pallas as pl
from jax.experimental.pallas import tpu as pltpu
from jax.experimental.pallas import tpu_sc as plsc

NC = 2
NS = 16
LANES = 16

N = 10000
D = 128
E = 320000

NW = NC * NS
EW = E // NW
C = 40
NCH = EW // C

EW2 = E // NS
NCH2 = EW2 // C

RB = 1000
NRW = N // RB
DBLK = 2000

_f32 = jnp.float32


def _deg_body(dst2_hbm, ew2_hbm, degp_hbm, deg_hbm,
              didx_all, wv_all, degv, dbuf, tmp2, sem):
    c = lax.axis_index("c")
    s = lax.axis_index("s")

    @pl.when(c == 0)
    def _():
        def zero16(i, _):
            degv[pl.ds(i * LANES, LANES)] = jnp.zeros((LANES,), _f32)
            return 0
        lax.fori_loop(0, N // LANES, zero16, 0)

        pltpu.sync_copy(dst2_hbm.at[s], didx_all)
        pltpu.sync_copy(ew2_hbm.at[s], wv_all)

        def grp(i, _):
            sl = pl.ds(i * LANES, LANES)
            plsc.addupdate_scatter(degv, [didx_all[sl]], wv_all[sl])
            return 0
        lax.fori_loop(0, EW2 // LANES, grp, 0)
        pltpu.sync_copy(degv, degp_hbm.at[pl.ds(s * N, N)])

    plsc.subcore_barrier()

    @pl.when((c == 0) & (s < N // DBLK))
    def _():
        copies = [
            pltpu.async_copy(degp_hbm.at[pl.ds(p * N + s * DBLK, DBLK)],
                             tmp2.at[p], sem)
            for p in range(NS)
        ]
        for cp in copies:
            cp.wait()

        def addv(i, _):
            sl = pl.ds(i * LANES, LANES)
            acc = tmp2[0, sl]
            for p in range(1, NS):
                acc = acc + tmp2[p, sl]
            dbuf[sl] = acc
            return 0
        lax.fori_loop(0, DBLK // LANES, addv, 0)
        pltpu.sync_copy(dbuf, deg_hbm.at[pl.ds(s * DBLK, DBLK)])


_SC_PARAMS = pltpu.CompilerParams(
    use_tc_tiling_on_sc=False, needs_layout_passes=False)

_sc_deg = pl.kernel(
    _deg_body,
    out_type=(jax.ShapeDtypeStruct((NS * N,), _f32),
              jax.ShapeDtypeStruct((N,), _f32)),
    compiler_params=_SC_PARAMS,
    mesh=plsc.VectorSubcoreMesh(core_axis_name="c", subcore_axis_name="s"),
    scratch_types=[
        pltpu.VMEM((EW2,), jnp.int32),
        pltpu.VMEM((EW2,), _f32),
        pltpu.VMEM((N,), _f32),
        pltpu.VMEM((DBLK,), _f32),
        pltpu.VMEM((NS, DBLK), _f32),
        pltpu.SemaphoreType.DMA,
    ],
)


def _spmm_body(zb_hbm, src3_hbm, dst3_hbm, ew3_hbm, zeros_hbm, out_hbm,
               sidxr, didx_all, wv_all,
               rows0, rows1, rows2, rows3, msg0, msg1, msg2, msg3,
               acc_sh,
               gsem0, gsem1, gsem2, gsem3,
               ssem0, ssem1, ssem2, ssem3,
               sisem0, sisem1, sisem2, sisem3):
    c = lax.axis_index("c")
    s = lax.axis_index("s")
    wid = c * NS + s
    rows = (rows0, rows1, rows2, rows3)
    msg = (msg0, msg1, msg2, msg3)
    gsem = (gsem0, gsem1, gsem2, gsem3)
    ssem = (ssem0, ssem1, ssem2, ssem3)
    sisem = (sisem0, sisem1, sisem2, sisem3)

    @pl.when(s < NRW)
    def _():
        pltpu.sync_copy(zeros_hbm.at[pl.ds(s * RB, RB)],
                        acc_sh.at[pl.ds(s * RB, RB)])
    pltpu.sync_copy(dst3_hbm.at[wid], didx_all)
    pltpu.sync_copy(ew3_hbm.at[wid], wv_all)
    plsc.subcore_barrier()

    def pre_s(x, sl):
        pltpu.async_copy(src3_hbm.at[wid, x], sidxr.at[sl], sisem[sl])

    def wait_s(sl):
        pltpu.make_async_copy(src3_hbm.at[wid, 0], sidxr.at[sl],
                              sisem[sl]).wait()

    def gather(isl, sl):
        pltpu.async_copy(zb_hbm.at[sidxr.at[isl]], rows[sl], gsem[sl])

    def gwait(sl):
        pltpu.make_async_copy(zb_hbm.at[sidxr.at[0]], rows[sl],
                              gsem[sl]).wait()

    def scat(ci, mi):
        pltpu.async_copy(msg[mi], acc_sh.at[didx_all.at[ci]], ssem[mi],
                         add=True)

    def swait(mi):
        pltpu.make_async_copy(msg[mi], acc_sh.at[didx_all.at[0]],
                              ssem[mi]).wait()

    def scale(ci, sl, mi):
        buf = rows[sl]
        m = msg[mi]
        cvec = jnp.full((LANES,), ci, jnp.int32)

        @plsc.parallel_loop(0, C, unroll=8)
        def _(r):
            wb = plsc.load_gather(
                wv_all, [cvec, jnp.full((LANES,), r, jnp.int32)])
            for g in range(D // 32):
                x32 = buf[r, pl.ds(32 * g, 32)]
                a, b = plsc.unpack(x32, format=plsc.PackFormat.INTERLEAVED)
                m[r, pl.ds(32 * g, LANES)] = a * wb
                m[r, pl.ds(32 * g + LANES, LANES)] = b * wb

    for x in range(4):
        pre_s(x, x)
    for x in range(3):
        wait_s(x)
        gather(x, x)

    def step(k, _):
        for off in range(4):
            ci = 4 * k + off
            gwait(off)
            if off >= 2:
                @pl.when(ci + 4 < NCH)
                def _(ci=ci, off=off):
                    pre_s(ci + 4, off)
            else:
                pre_s(ci + 4, off)
            if off == 3:
                @pl.when(ci + 3 < NCH)
                def _(ci=ci, off=off):
                    wait_s((off + 3) % 4)
                    gather((off + 3) % 4, (off + 3) % 4)
            else:
                wait_s((off + 3) % 4)
                gather((off + 3) % 4, (off + 3) % 4)

            @pl.when(k > 0)
            def _(off=off):
                swait(off)
            scale(ci, off, off)
            scat(ci, off)
        return 0
    lax.fori_loop(0, NCH // 4, step, 0)

    for ci, off in ((NCH - 2, 0), (NCH - 1, 1)):
        gwait(off)
        swait(off)
        scale(ci, off, off)
        scat(ci, off)
    swait(2)
    swait(3)
    swait(0)
    swait(1)

    plsc.subcore_barrier()

    @pl.when(s < NRW)
    def _():
        pltpu.sync_copy(acc_sh.at[pl.ds(s * RB, RB)],
                        out_hbm.at[pl.ds(c * N + s * RB, RB)])


assert NCH % 4 == 2

_sc_spmm = pl.kernel(
    _spmm_body,
    out_type=jax.ShapeDtypeStruct((NC * N, D), _f32),
    compiler_params=_SC_PARAMS,
    mesh=plsc.VectorSubcoreMesh(core_axis_name="c", subcore_axis_name="s"),
    scratch_types=[
        pltpu.VMEM((4, C), jnp.int32),
        pltpu.VMEM((NCH, C), jnp.int32),
        pltpu.VMEM((NCH, C), _f32),
        pltpu.VMEM((C, D), jnp.bfloat16),
        pltpu.VMEM((C, D), jnp.bfloat16),
        pltpu.VMEM((C, D), jnp.bfloat16),
        pltpu.VMEM((C, D), jnp.bfloat16),
        pltpu.VMEM((C, D), _f32),
        pltpu.VMEM((C, D), _f32),
        pltpu.VMEM((C, D), _f32),
        pltpu.VMEM((C, D), _f32),
        pltpu.VMEM_SHARED((N, D), _f32),
    ] + [pltpu.SemaphoreType.DMA] * 12,
)


BR = 5000
G = N // BR


def _tc1_body(deg_ref, x_ref, W_ref, Wp_ref, Wr_ref, br_ref,
              z_ref, zb_ref, id_ref):
    dis = lax.rsqrt(deg_ref[...] + 1.0)
    xb = x_ref[...]
    z_ref[...] = jnp.dot(xb, W_ref[...], preferred_element_type=_f32) * dis
    zb_ref[...] = (jnp.dot(xb, Wp_ref[...], preferred_element_type=_f32)
                   * dis).astype(jnp.bfloat16)
    id_ref[...] = jnp.dot(xb, Wr_ref[...], preferred_element_type=_f32) + br_ref[...]


def _ln(h, g, be):
    mu = jnp.mean(h, axis=-1, keepdims=True)
    var = jnp.mean((h - mu) ** 2, axis=-1, keepdims=True)
    return (h - mu) * lax.rsqrt(var + 1e-5) * g + be


def _tc_post_body(acc_ref, z_ref, id_ref, deg_ref, b_ref, g_ref, be_ref,
                  Wn_ref, Wpn_ref, Wrn_ref, brn_ref,
                  zn_ref, zbn_ref, idn_ref):
    dis = lax.rsqrt(deg_ref[...] + 1.0)
    out = (acc_ref[0] + acc_ref[1] + z_ref[...]) * dis + b_ref[...] + id_ref[...]
    h = _ln(out, g_ref[...], be_ref[...])
    h = h * jax.nn.sigmoid(h)
    zn_ref[...] = jnp.dot(h, Wn_ref[...], preferred_element_type=_f32) * dis
    zbn_ref[...] = (jnp.dot(h, Wpn_ref[...], preferred_element_type=_f32)
                    * dis).astype(jnp.bfloat16)
    idn_ref[...] = jnp.dot(h, Wrn_ref[...], preferred_element_type=_f32) + brn_ref[...]


def _tc_final_body(acc_ref, z_ref, id_ref, deg_ref, b_ref, g_ref, be_ref,
                   res_ref):
    i = pl.program_id(0)
    dis = lax.rsqrt(deg_ref[...] + 1.0)
    out = (acc_ref[0] + acc_ref[1] + z_ref[...]) * dis + b_ref[...] + id_ref[...]
    h = _ln(out, g_ref[...], be_ref[...])
    part = jnp.sum(h, axis=0, keepdims=True) * (1.0 / N)

    @pl.when(i == 0)
    def _():
        res_ref[...] = jnp.zeros_like(res_ref)
    res_ref[...] += part


def _row_spec(last):
    return pl.BlockSpec((BR, last), lambda i: (i, 0))


def _full_spec(shape):
    nd = len(shape)
    return pl.BlockSpec(shape, lambda i: (0,) * nd)


def _tc1(deg, x, W, Wp, Wr, br):
    return pl.pallas_call(
        _tc1_body,
        grid=(G,),
        in_specs=[_row_spec(1), _row_spec(D), _full_spec((D, D)),
                  _full_spec((D, D)), _full_spec((D, D)), _full_spec((1, D))],
        out_specs=[_row_spec(D), _row_spec(D), _row_spec(D)],
        out_shape=[jax.ShapeDtypeStruct((N, D), _f32),
                   jax.ShapeDtypeStruct((N, D), jnp.bfloat16),
                   jax.ShapeDtypeStruct((N, D), _f32)],
    )(deg, x, W, Wp, Wr, br)


def _tc_post(acc, z, idn, deg, b, g, be, Wn, Wpn, Wrn, brn):
    return pl.pallas_call(
        _tc_post_body,
        grid=(G,),
        in_specs=[pl.BlockSpec((NC, BR, D), lambda i: (0, i, 0)),
                  _row_spec(D), _row_spec(D), _row_spec(1),
                  _full_spec((1, D)), _full_spec((1, D)), _full_spec((1, D)),
                  _full_spec((D, D)), _full_spec((D, D)), _full_spec((D, D)),
                  _full_spec((1, D))],
        out_specs=[_row_spec(D), _row_spec(D), _row_spec(D)],
        out_shape=[jax.ShapeDtypeStruct((N, D), _f32),
                   jax.ShapeDtypeStruct((N, D), jnp.bfloat16),
                   jax.ShapeDtypeStruct((N, D), _f32)],
    )(acc, z, idn, deg, b, g, be, Wn, Wpn, Wrn, brn)


def _tc_final(acc, z, idn, deg, b, g, be):
    return pl.pallas_call(
        _tc_final_body,
        grid=(G,),
        in_specs=[pl.BlockSpec((NC, BR, D), lambda i: (0, i, 0)),
                  _row_spec(D), _row_spec(D), _row_spec(1),
                  _full_spec((1, D)), _full_spec((1, D)), _full_spec((1, D))],
        out_specs=pl.BlockSpec((1, D), lambda i: (0, 0)),
        out_shape=jax.ShapeDtypeStruct((1, D), _f32),
    )(acc, z, idn, deg, b, g, be)


def kernel(x, edge_index, edge_weight,
           W1, b1, Wr1, br1, g1, be1,
           W2, b2, Wr2, br2, g2, be2,
           W3, b3, Wr3, br3, g3, be3):
    src = edge_index[0].astype(jnp.int32)
    dst = edge_index[1].astype(jnp.int32)
    ew = edge_weight
    zeros = jnp.zeros((N, D), _f32)

    src3 = src.reshape(NW, NCH, C)
    dst3 = dst.reshape(NW, NCH, C)
    ew3 = ew.reshape(NW, NCH, C)
    dst2 = dst.reshape(NS, EW2)
    ew2 = ew.reshape(NS, EW2)

    _, deg_raw = _sc_deg(dst2, ew2)
    deg = deg_raw.reshape(N, 1)

    def r1(v):
        return v.reshape(1, D)

    def spmm(zb):
        return _sc_spmm(zb, src3, dst3, ew3, zeros).reshape(NC, N, D)

    perm = np.empty((D,), np.int32)
    for g in range(D // 32):
        for i in range(16):
            perm[32 * g + 2 * i] = 32 * g + i
            perm[32 * g + 2 * i + 1] = 32 * g + 16 + i
    W1p, W2p, W3p = W1[:, perm], W2[:, perm], W3[:, perm]

    z, zb, idn = _tc1(deg, x, W1, W1p, Wr1, r1(br1))
    acc = spmm(zb)
    z, zb, idn = _tc_post(acc, z, idn, deg, r1(b1), r1(g1), r1(be1),
                          W2, W2p, Wr2, r1(br2))
    acc = spmm(zb)
    z, zb, idn = _tc_post(acc, z, idn, deg, r1(b2), r1(g2), r1(be2),
                          W3, W3p, Wr3, r1(br3))
    acc = spmm(zb)
    return _tc_final(acc, z, idn, deg, r1(b3), r1(g3), r1(be3))
